# Initial kernel scaffold; baseline (speedup 1.0000x reference)
import jax
import jax.numpy as jnp
from jax.experimental import pallas as pl


def kernel(x, table, W, b):
    raise NotImplementedError("write your pallas kernel here")


# speedup vs baseline: 4.9532x; 4.9532x over previous
import jax
import jax.numpy as jnp
from jax import lax
from jax.experimental import pallas as pl
from jax.experimental.pallas import tpu as pltpu
from jax.experimental.pallas import tpu_sc as plsc

_NC, _NS, _L = 2, 16, 16
_NW = _NC * _NS

_B = 16384 * 200
_PER_W = _B // _NW
_CHUNK = 2048
_NCHUNK = _PER_W // _CHUNK
_OUT_CHUNK = _CHUNK * 4


def _sc_body(x_hbm, tab_hbm, w_hbm, b_hbm, out_hbm,
             tab_v, w_v, b_v, fused_v, x_vs, out_vs, in_sems, out_sems):
    wid = lax.axis_index("s") * _NC + lax.axis_index("c")
    base = pl.multiple_of(wid * _PER_W, _CHUNK)

    pltpu.sync_copy(tab_hbm, tab_v)
    pltpu.sync_copy(w_hbm, w_v)
    pltpu.sync_copy(b_hbm, b_v)

    iota = lax.iota(jnp.int32, _L)
    rep4 = lax.shift_right_logical(iota, 2)
    mod4 = jnp.bitwise_and(iota, 3)

    for t in range(3):
        j = iota + (16 * t)
        k4 = lax.shift_left(lax.shift_right_logical(j, 2), 2)
        c = jnp.bitwise_and(j, 3)
        acc = plsc.load_gather(b_v, [c])
        for d in range(4):
            tv = plsc.load_gather(tab_v, [k4 + d])
            wv = plsc.load_gather(w_v, [lax.shift_left(c, 2) + d])
            acc = acc + tv * wv
        fused_v[pl.ds(16 * t, 16)] = acc

    def in_copy(g, s):
        off = pl.multiple_of(base + g * _CHUNK, _CHUNK)
        return pltpu.make_async_copy(x_hbm.at[pl.ds(off, _CHUNK)],
                                     x_vs[s], in_sems[s])

    def out_copy(g, s):
        off = pl.multiple_of((base + g * _CHUNK) * 4, _OUT_CHUNK)
        return pltpu.make_async_copy(out_vs[s],
                                     out_hbm.at[pl.ds(off, _OUT_CHUNK)],
                                     out_sems[s])

    def compute(s):
        x_v, out_v = x_vs[s], out_vs[s]

        def body(o, _):
            o16 = pl.multiple_of(o * 16, 16)
            idx_rep = plsc.load_gather(x_v, [o * 4 + rep4])
            gidx = jnp.bitwise_or(lax.shift_left(idx_rep, 2), mod4)
            out_v[pl.ds(o16, 16)] = plsc.load_gather(fused_v, [gidx])
            return ()
        lax.fori_loop(0, _CHUNK // 4, body, (), unroll=4)

    in_copy(0, 0).start()

    def step(h, _):
        for s in range(2):
            g = h + s

            @pl.when(g + 1 < _NCHUNK)
            def _():
                in_copy(g + 1, 1 - s).start()

            in_copy(g, s).wait()

            @pl.when(g >= 2)
            def _():
                out_copy(g - 2, s).wait()

            compute(s)
            out_copy(g, s).start()
        return ()

    lax.fori_loop(0, _NCHUNK // 2, lambda i, c: step(i * 2, c), ())
    out_copy(_NCHUNK - 2, 0).wait()
    out_copy(_NCHUNK - 1, 1).wait()


@jax.jit
def _run(x_flat, tab_p, w_p, b_p):
    mesh = plsc.VectorSubcoreMesh(core_axis_name="c", subcore_axis_name="s",
                                  num_cores=_NC, num_subcores=_NS)
    f = pl.kernel(
        _sc_body,
        out_type=jax.ShapeDtypeStruct((_B * 4,), jnp.float32),
        mesh=mesh,
        compiler_params=pltpu.CompilerParams(needs_layout_passes=False),
        scratch_types=[
            pltpu.VMEM((64,), jnp.float32),
            pltpu.VMEM((16,), jnp.float32),
            pltpu.VMEM((16,), jnp.float32),
            pltpu.VMEM((64,), jnp.float32),
            [pltpu.VMEM((_CHUNK,), jnp.int32),
             pltpu.VMEM((_CHUNK,), jnp.int32)],
            [pltpu.VMEM((_OUT_CHUNK,), jnp.float32),
             pltpu.VMEM((_OUT_CHUNK,), jnp.float32)],
            [pltpu.SemaphoreType.DMA, pltpu.SemaphoreType.DMA],
            [pltpu.SemaphoreType.DMA, pltpu.SemaphoreType.DMA],
        ],
    )
    return f(x_flat, tab_p, w_p, b_p)


def kernel(x, table, W, b):
    x_flat = x.reshape(-1).astype(jnp.int32)
    tab_p = jnp.zeros((64,), jnp.float32).at[:40].set(table.reshape(-1))
    w_p = W.reshape(-1).astype(jnp.float32)
    b_p = jnp.zeros((16,), jnp.float32).at[:4].set(b)
    out_flat = _run(x_flat, tab_p, w_p, b_p)
    return out_flat.reshape(x.shape[0], x.shape[1], 4)


# speedup vs baseline: 5.4639x; 1.1031x over previous
import jax
import jax.numpy as jnp
from jax import lax
from jax.experimental import pallas as pl
from jax.experimental.pallas import tpu as pltpu
from jax.experimental.pallas import tpu_sc as plsc

_NC, _NS, _L = 2, 16, 16
_NW = _NC * _NS

_B = 16384 * 200
_PER_W = _B // _NW
_CHUNK = 2048
_NCHUNK = _PER_W // _CHUNK
_OUT_CHUNK = _CHUNK * 4


def _sc_body(x_hbm, tab_hbm, w_hbm, b_hbm, out_hbm,
             tab_v, w_v, b_v, fused_v, x_vs, out_vs, in_sems, out_sems):
    wid = lax.axis_index("s") * _NC + lax.axis_index("c")
    base = pl.multiple_of(wid * _PER_W, _CHUNK)

    pltpu.sync_copy(tab_hbm, tab_v)
    pltpu.sync_copy(w_hbm, w_v)
    pltpu.sync_copy(b_hbm, b_v)

    iota = lax.iota(jnp.int32, _L)
    rep4 = lax.shift_right_logical(iota, 2)
    mod4 = jnp.bitwise_and(iota, 3)

    for t in range(3):
        j = iota + (16 * t)
        k4 = lax.shift_left(lax.shift_right_logical(j, 2), 2)
        c = jnp.bitwise_and(j, 3)
        acc = plsc.load_gather(b_v, [c])
        for d in range(4):
            tv = plsc.load_gather(tab_v, [k4 + d])
            wv = plsc.load_gather(w_v, [lax.shift_left(c, 2) + d])
            acc = acc + tv * wv
        fused_v[pl.ds(16 * t, 16)] = acc

    def in_copy(g, s):
        off = pl.multiple_of(base + g * _CHUNK, _CHUNK)
        return pltpu.make_async_copy(x_hbm.at[pl.ds(off, _CHUNK)],
                                     x_vs[s], in_sems[s])

    def out_copy(g, s):
        off = pl.multiple_of((base + g * _CHUNK) * 4, _OUT_CHUNK)
        return pltpu.make_async_copy(out_vs[s],
                                     out_hbm.at[pl.ds(off, _OUT_CHUNK)],
                                     out_sems[s])

    def compute(s):
        x_v, out_v = x_vs[s], out_vs[s]

        @plsc.parallel_loop(0, _CHUNK // 4, unroll=8)
        def body(o):
            o16 = pl.multiple_of(o * 16, 16)
            idx_rep = plsc.load_gather(x_v, [o * 4 + rep4])
            gidx = jnp.bitwise_or(lax.shift_left(idx_rep, 2), mod4)
            out_v[pl.ds(o16, 16)] = plsc.load_gather(fused_v, [gidx])

    in_copy(0, 0).start()

    def step(h, _):
        for s in range(2):
            g = h + s

            @pl.when(g + 1 < _NCHUNK)
            def _():
                in_copy(g + 1, 1 - s).start()

            in_copy(g, s).wait()

            @pl.when(g >= 2)
            def _():
                out_copy(g - 2, s).wait()

            compute(s)
            out_copy(g, s).start()
        return ()

    lax.fori_loop(0, _NCHUNK // 2, lambda i, c: step(i * 2, c), ())
    out_copy(_NCHUNK - 2, 0).wait()
    out_copy(_NCHUNK - 1, 1).wait()


@jax.jit
def _run(x_flat, tab_p, w_p, b_p):
    mesh = plsc.VectorSubcoreMesh(core_axis_name="c", subcore_axis_name="s",
                                  num_cores=_NC, num_subcores=_NS)
    f = pl.kernel(
        _sc_body,
        out_type=jax.ShapeDtypeStruct((_B * 4,), jnp.float32),
        mesh=mesh,
        compiler_params=pltpu.CompilerParams(needs_layout_passes=False),
        scratch_types=[
            pltpu.VMEM((64,), jnp.float32),
            pltpu.VMEM((16,), jnp.float32),
            pltpu.VMEM((16,), jnp.float32),
            pltpu.VMEM((64,), jnp.float32),
            [pltpu.VMEM((_CHUNK,), jnp.int32),
             pltpu.VMEM((_CHUNK,), jnp.int32)],
            [pltpu.VMEM((_OUT_CHUNK,), jnp.float32),
             pltpu.VMEM((_OUT_CHUNK,), jnp.float32)],
            [pltpu.SemaphoreType.DMA, pltpu.SemaphoreType.DMA],
            [pltpu.SemaphoreType.DMA, pltpu.SemaphoreType.DMA],
        ],
    )
    return f(x_flat, tab_p, w_p, b_p)


def kernel(x, table, W, b):
    x_flat = x.reshape(-1).astype(jnp.int32)
    tab_p = jnp.zeros((64,), jnp.float32).at[:40].set(table.reshape(-1))
    w_p = W.reshape(-1).astype(jnp.float32)
    b_p = jnp.zeros((16,), jnp.float32).at[:4].set(b)
    out_flat = _run(x_flat, tab_p, w_p, b_p)
    return out_flat.reshape(x.shape[0], x.shape[1], 4)


# speedup vs baseline: 135.0050x; 24.7085x over previous
import jax
import jax.numpy as jnp
from jax import lax
from jax.experimental import pallas as pl
from jax.experimental.pallas import tpu as pltpu
from jax.experimental.pallas import tpu_sc as plsc

_NC, _NS, _L = 2, 16, 16
_NW = _NC * _NS

_NI = 16384
_NJ = 200
_IB = _NI // 128
_R = 2
_NG = _IB // _R
_SPAN = _R * 512
_NUNIT = (_NJ // 8) * _NG
_PER_W = _NUNIT // _NW


def _sc_body(x_hbm, tab_hbm, w_hbm, b_hbm, out_hbm,
             tab_v, w_v, b_v, fused_cs, x_vs, out_vs, in_sems, out_sems):
    wid = lax.axis_index("s") * _NC + lax.axis_index("c")
    u0 = wid * _PER_W

    pltpu.sync_copy(tab_hbm, tab_v)
    pltpu.sync_copy(w_hbm, w_v)
    pltpu.sync_copy(b_hbm, b_v)

    iota = lax.iota(jnp.int32, _L)
    zero16 = jnp.bitwise_and(iota, 0)

    k4 = lax.shift_left(iota, 2)
    for c in range(4):
        acc = plsc.load_gather(b_v, [zero16 + (c + 1)])
        for d in range(4):
            tv = plsc.load_gather(tab_v, [k4 + d])
            wv = plsc.load_gather(w_v, [zero16 + (4 * c + d + 1)])
            acc = acc + tv * wv
        fused_cs[c][...] = acc

    def in_copy(u, s):
        jb = lax.shift_right_logical(u, 6)
        ib0 = pl.multiple_of(lax.shift_left(jnp.bitwise_and(u, 63), 1), _R)
        return pltpu.make_async_copy(
            x_hbm.at[jb, pl.ds(ib0, _R)], x_vs[s], in_sems[s])

    def out_copy_jr(u, s, jr):
        jb = lax.shift_right_logical(u, 6)
        ib0 = lax.shift_left(jnp.bitwise_and(u, 63), 1)
        off = pl.multiple_of((jb * 8 + jr) * 65536 + ib0 * 512, _SPAN)
        return pltpu.make_async_copy(
            out_vs[s].at[jr], out_hbm.at[pl.ds(off, _SPAN)], out_sems[s])

    def compute(s):
        x_v, out_v = x_vs[s], out_vs[s]

        @plsc.parallel_loop(0, 8, unroll=2)
        def body(m):
            m16 = pl.multiple_of(m * 16, 16)
            for r in range(_R):
                for jr in range(8):
                    xv = x_v[r, jr, pl.ds(m16, 16)]
                    for c in range(4):
                        out_v[jr, pl.ds(r * 512 + c * 128 + m16, 16)] = (
                            plsc.load_gather(fused_cs[c], [xv]))

    in_copy(u0, 0).start()

    def step(h, _):
        for s in range(2):
            t = h + s
            u = u0 + t

            @pl.when(t + 1 < _PER_W)
            def _():
                in_copy(u + 1, 1 - s).start()

            in_copy(u, s).wait()

            @pl.when(t >= 2)
            def _():
                for jr in range(8):
                    out_copy_jr(u - 2, s, jr).wait()

            compute(s)
            for jr in range(8):
                out_copy_jr(u, s, jr).start()
        return ()

    lax.fori_loop(0, _PER_W // 2, lambda i, c: step(i * 2, c), ())
    for jr in range(8):
        out_copy_jr(u0 + _PER_W - 2, 0, jr).wait()
        out_copy_jr(u0 + _PER_W - 1, 1, jr).wait()


@jax.jit
def _run(x4, tab_p, w_p, b_p):
    mesh = plsc.VectorSubcoreMesh(core_axis_name="c", subcore_axis_name="s",
                                  num_cores=_NC, num_subcores=_NS)
    f = pl.kernel(
        _sc_body,
        out_type=jax.ShapeDtypeStruct((_NI * _NJ * 4,), jnp.float32),
        mesh=mesh,
        compiler_params=pltpu.CompilerParams(needs_layout_passes=False),
        scratch_types=[
            pltpu.VMEM((64,), jnp.float32),
            pltpu.VMEM((32,), jnp.float32),
            pltpu.VMEM((16,), jnp.float32),
            [pltpu.VMEM((16,), jnp.float32) for _ in range(4)],
            [pltpu.VMEM((_R, 8, 128), jnp.int32),
             pltpu.VMEM((_R, 8, 128), jnp.int32)],
            [pltpu.VMEM((8, _SPAN), jnp.float32),
             pltpu.VMEM((8, _SPAN), jnp.float32)],
            [pltpu.SemaphoreType.DMA, pltpu.SemaphoreType.DMA],
            [pltpu.SemaphoreType.DMA, pltpu.SemaphoreType.DMA],
        ],
    )
    return f(x4, tab_p, w_p, b_p)


def kernel(x, table, W, b):
    x4 = (jnp.transpose(x.astype(jnp.int32))
          .reshape(25, 8, 128, 128).transpose(0, 2, 1, 3))
    tab_p = jnp.zeros((64,), jnp.float32).at[:40].set(table.reshape(-1))
    w_p = jnp.zeros((32,), jnp.float32).at[1:17].set(W.reshape(-1))
    b_p = jnp.zeros((16,), jnp.float32).at[1:5].set(b)
    out_flat = _run(x4, tab_p, w_p, b_p)
    return (out_flat.reshape(_NJ, _IB, 4, 128)
            .transpose(1, 3, 0, 2).reshape(_NI, _NJ, 4))


# speedup vs baseline: 190.3855x; 1.4102x over previous
import jax
import jax.numpy as jnp
from jax import lax
from jax.experimental import pallas as pl
from jax.experimental.pallas import tpu as pltpu
from jax.experimental.pallas import tpu_sc as plsc

_NC, _NS, _L = 2, 16, 16
_NW = _NC * _NS

_NI = 16384
_NJ = 200
_IB = _NI // 128
_R = 2
_NG = _IB // _R
_SPAN = _R * 512
_NUNIT = (_NJ // 8) * _NG
_PER_W = _NUNIT // _NW


def _sc_body(x_hbm, tab_hbm, w_hbm, b_hbm, out_hbm,
             tab_v, w_v, b_v, fused_cs, x_vs, out_vs, in_sems, out_sems):
    wid = lax.axis_index("s") * _NC + lax.axis_index("c")
    u0 = wid * _PER_W

    pltpu.sync_copy(tab_hbm, tab_v)
    pltpu.sync_copy(w_hbm, w_v)
    pltpu.sync_copy(b_hbm, b_v)

    iota = lax.iota(jnp.int32, _L)
    zero16 = jnp.bitwise_and(iota, 0)

    k4 = lax.shift_left(iota, 2)
    for c in range(4):
        acc = plsc.load_gather(b_v, [zero16 + (c + 1)])
        for d in range(4):
            tv = plsc.load_gather(tab_v, [k4 + d])
            wv = plsc.load_gather(w_v, [zero16 + (4 * c + d + 1)])
            acc = acc + tv * wv
        fused_cs[c][...] = acc

    def in_copy(u, s):
        jb = lax.shift_right_logical(u, 6)
        ib0 = pl.multiple_of(lax.shift_left(jnp.bitwise_and(u, 63), 1), _R)
        return pltpu.make_async_copy(
            x_hbm.at[jb, pl.ds(ib0, _R)], x_vs[s], in_sems[s])

    def out_copy_jr(u, s, jr):
        jb = lax.shift_right_logical(u, 6)
        ib0 = lax.shift_left(jnp.bitwise_and(u, 63), 1)
        off = pl.multiple_of((jb * 8 + jr) * 65536 + ib0 * 512, _SPAN)
        return pltpu.make_async_copy(
            out_vs[s].at[jr], out_hbm.at[pl.ds(off, _SPAN)], out_sems[s])

    def compute(s):
        x_v, out_v = x_vs[s], out_vs[s]

        @plsc.parallel_loop(0, 8, unroll=4)
        def body(m):
            m16 = pl.multiple_of(m * 16, 16)
            for r in range(_R):
                for jr in range(8):
                    xv = x_v[r, jr, pl.ds(m16, 16)]
                    vals = [plsc.load_gather(fused_cs[c], [xv])
                            for c in range(4)]
                    for c in range(4):
                        out_v[jr, pl.ds(r * 512 + c * 128 + m16, 16)] = vals[c]

    in_copy(u0, 0).start()

    def step(h, _):
        for s in range(2):
            t = h + s
            u = u0 + t

            @pl.when(t + 1 < _PER_W)
            def _():
                in_copy(u + 1, 1 - s).start()

            in_copy(u, s).wait()

            @pl.when(t >= 2)
            def _():
                for jr in range(8):
                    out_copy_jr(u - 2, s, jr).wait()

            compute(s)
            for jr in range(8):
                out_copy_jr(u, s, jr).start()
        return ()

    lax.fori_loop(0, _PER_W // 2, lambda i, c: step(i * 2, c), ())
    for jr in range(8):
        out_copy_jr(u0 + _PER_W - 2, 0, jr).wait()
        out_copy_jr(u0 + _PER_W - 1, 1, jr).wait()


@jax.jit
def _run(x4, tab_p, w_p, b_p):
    mesh = plsc.VectorSubcoreMesh(core_axis_name="c", subcore_axis_name="s",
                                  num_cores=_NC, num_subcores=_NS)
    f = pl.kernel(
        _sc_body,
        out_type=jax.ShapeDtypeStruct((_NI * _NJ * 4,), jnp.float32),
        mesh=mesh,
        compiler_params=pltpu.CompilerParams(needs_layout_passes=False),
        scratch_types=[
            pltpu.VMEM((64,), jnp.float32),
            pltpu.VMEM((32,), jnp.float32),
            pltpu.VMEM((16,), jnp.float32),
            [pltpu.VMEM((16,), jnp.float32) for _ in range(4)],
            [pltpu.VMEM((_R, 8, 128), jnp.int32),
             pltpu.VMEM((_R, 8, 128), jnp.int32)],
            [pltpu.VMEM((8, _SPAN), jnp.float32),
             pltpu.VMEM((8, _SPAN), jnp.float32)],
            [pltpu.SemaphoreType.DMA, pltpu.SemaphoreType.DMA],
            [pltpu.SemaphoreType.DMA, pltpu.SemaphoreType.DMA],
        ],
    )
    return f(x4, tab_p, w_p, b_p)


def kernel(x, table, W, b):
    x4 = (jnp.transpose(x.astype(jnp.int32))
          .reshape(25, 8, 128, 128).transpose(0, 2, 1, 3))
    tab_p = jnp.zeros((64,), jnp.float32).at[:40].set(table.reshape(-1))
    w_p = jnp.zeros((32,), jnp.float32).at[1:17].set(W.reshape(-1))
    b_p = jnp.zeros((16,), jnp.float32).at[1:5].set(b)
    out_flat = _run(x4, tab_p, w_p, b_p)
    return (out_flat.reshape(_NJ, _IB, 4, 128)
            .transpose(1, 3, 0, 2).reshape(_NI, _NJ, 4))
